# Initial kernel scaffold; baseline (speedup 1.0000x reference)
import jax
import jax.numpy as jnp
from jax.experimental import pallas as pl


def kernel(x, edge_index, batch, params):
    raise NotImplementedError("write your pallas kernel here")


# speedup vs baseline: 4.2280x; 4.2280x over previous
import functools

import jax
import jax.numpy as jnp
from jax import lax
from jax.experimental import pallas as pl
from jax.experimental.pallas import tpu as pltpu
from jax.experimental.pallas import tpu_sc as plsc

NB = 16
TAU = 1.0
TW = 128
MW = 128
CH = 80
NBLK = 1000
EBLK = 8000


def _silu(t):
    return t / (1.0 + jnp.exp(-t))


def _pre_body(h_ref, pv_ref, ws_ref, wd_ref, ts_ref, td_ref):
    h = h_ref[...]
    pv = pv_ref[...]
    z = jnp.zeros((h.shape[0], TW - 52), jnp.float32)
    ts_ref[...] = jnp.concatenate([jnp.dot(h, ws_ref[...]), pv, z], axis=1)
    td_ref[...] = jnp.concatenate([jnp.dot(h, wd_ref[...]), -pv, z], axis=1)


def _pre_tables(h, pv, w_src, w_dst):
    n, cin = h.shape
    grid = (n // NBLK,)
    return pl.pallas_call(
        _pre_body,
        grid=grid,
        in_specs=[
            pl.BlockSpec((NBLK, cin), lambda i: (i, 0)),
            pl.BlockSpec((NBLK, 4), lambda i: (i, 0)),
            pl.BlockSpec((cin, 48), lambda i: (0, 0)),
            pl.BlockSpec((cin, 48), lambda i: (0, 0)),
        ],
        out_specs=[
            pl.BlockSpec((NBLK, TW), lambda i: (i, 0)),
            pl.BlockSpec((NBLK, TW), lambda i: (i, 0)),
        ],
        out_shape=[
            jax.ShapeDtypeStruct((n, TW), jnp.float32),
            jax.ShapeDtypeStruct((n, TW), jnp.float32),
        ],
    )(h, pv, w_src, w_dst)


def _sc_gather(t_src, t_dst, src, dst):
    e = src.shape[0]
    mesh = plsc.VectorSubcoreMesh(core_axis_name="c", subcore_axis_name="s")
    info = plsc.get_sparse_core_info()
    nw = info.num_cores * info.num_subcores
    per = e // nw
    nch = per // CH

    @functools.partial(
        pl.kernel,
        out_type=jax.ShapeDtypeStruct((e, TW), jnp.float32),
        mesh=mesh,
        scratch_types=[
            pltpu.VMEM((CH,), jnp.int32),
            pltpu.VMEM((CH,), jnp.int32),
            pltpu.VMEM((CH, TW), jnp.float32),
            pltpu.VMEM((CH, TW), jnp.float32),
            pltpu.SemaphoreType.DMA,
            pltpu.SemaphoreType.DMA,
        ],
    )
    def gat(ts_hbm, td_hbm, src_hbm, dst_hbm, out_hbm, isv, idv, gs, gd, s1, s2):
        wid = lax.axis_index("s") * info.num_cores + lax.axis_index("c")
        base = wid * per

        def chunk(i, _):
            off = pl.multiple_of(base + i * CH, 16)
            pltpu.sync_copy(src_hbm.at[pl.ds(off, CH)], isv)
            pltpu.sync_copy(dst_hbm.at[pl.ds(off, CH)], idv)
            c1 = pltpu.async_copy(ts_hbm.at[isv], gs, s1)
            c2 = pltpu.async_copy(td_hbm.at[idv], gd, s2)
            c1.wait()
            c2.wait()

            def radd(j, _):
                for k in range(TW // 16):
                    sl = pl.ds(k * 16, 16)
                    gs[j, sl] = gs[j, sl] + gd[j, sl]
                return 0

            lax.fori_loop(0, CH, radd, 0)
            pltpu.sync_copy(gs, out_hbm.at[pl.ds(off, CH)])
            return 0

        lax.fori_loop(0, nch, chunk, 0)

    return gat(t_src, t_dst, src, dst)


def _edge_body(e_ref, w_ref, m_ref):
    ev = e_ref[...]
    w = w_ref[...]
    rel = ev[:, 48:50]
    relv = ev[:, 50:52]
    dist = jnp.sum(rel * rel, axis=1, keepdims=True)
    dvr = jnp.sum(relv * rel, axis=1, keepdims=True)
    te = ev[:, :32] + dist * w[0:1, :] + dvr * w[1:2, :] + w[2:3, :]
    se = _silu(te)
    tv = ev[:, 32:48] + dist * w[3:4, :16] + dvr * w[4:5, :16] + w[5:6, :16]
    sv = _silu(tv)
    vw = jnp.sum(sv * w[6:7, :16], axis=1, keepdims=True) + w[7:8, 0:1]
    mv = vw * rel
    rb = ev.shape[0]
    m_ref[...] = jnp.concatenate(
        [se, mv, jnp.ones((rb, 1), jnp.float32),
         jnp.zeros((rb, MW - 35), jnp.float32)], axis=1)


def _edge_dense(esum, wpack):
    e = esum.shape[0]
    grid = (e // EBLK,)
    return pl.pallas_call(
        _edge_body,
        grid=grid,
        in_specs=[
            pl.BlockSpec((EBLK, TW), lambda i: (i, 0)),
            pl.BlockSpec((8, 32), lambda i: (0, 0)),
        ],
        out_specs=pl.BlockSpec((EBLK, MW), lambda i: (i, 0)),
        out_shape=jax.ShapeDtypeStruct((e, MW), jnp.float32),
    )(esum, wpack)


def _sc_scatter(m_rows, dst, zeros_n):
    e = dst.shape[0]
    n = zeros_n.shape[0]
    mesh = plsc.VectorSubcoreMesh(core_axis_name="c", subcore_axis_name="s")
    info = plsc.get_sparse_core_info()
    ncores = info.num_cores
    nsub = info.num_subcores
    nw = ncores * nsub
    per = e // nw
    nch = per // CH
    stripe = n // nsub

    @functools.partial(
        pl.kernel,
        out_type=jax.ShapeDtypeStruct((ncores, n, MW), jnp.float32),
        mesh=mesh,
        scratch_types=[
            pltpu.VMEM((CH,), jnp.int32),
            pltpu.VMEM((CH, MW), jnp.float32),
            pltpu.VMEM_SHARED((n, MW), jnp.float32),
        ],
    )
    def sca(m_hbm, dst_hbm, z_hbm, out_hbm, idxv, mv, acc):
        cid = lax.axis_index("c")
        sid = lax.axis_index("s")
        wid = sid * ncores + cid
        srow = pl.multiple_of(sid * stripe, 16)
        pltpu.sync_copy(z_hbm.at[pl.ds(srow, stripe)],
                        acc.at[pl.ds(srow, stripe)])
        plsc.subcore_barrier()

        def chunk(i, _):
            off = pl.multiple_of(wid * per + i * CH, 16)
            pltpu.sync_copy(dst_hbm.at[pl.ds(off, CH)], idxv)
            pltpu.sync_copy(m_hbm.at[pl.ds(off, CH)], mv)
            pltpu.sync_copy(mv, acc.at[idxv], add=True)
            return 0

        lax.fori_loop(0, nch, chunk, 0)
        plsc.subcore_barrier()
        pltpu.sync_copy(acc.at[pl.ds(srow, stripe)],
                        out_hbm.at[cid, pl.ds(srow, stripe)])

    return sca(m_rows, dst, zeros_n)


def _node_body(has_sc, h_ref, a0_ref, a1_ref, we2_ref, whx_ref, whm_ref,
               wh2_ref, wsc_ref, v_ref, o_ref):
    h = h_ref[...]
    acc = a0_ref[...] + a1_ref[...]
    v = v_ref[...]
    s_sum = acc[:, :32]
    m_v = acc[:, 32:34]
    deg = acc[:, 34:35]
    m_h = jnp.dot(s_sum, we2_ref[...]) + deg * v[0:1, :16]
    norm = jnp.sqrt(jnp.sum(m_v * m_v, axis=1, keepdims=True) + 1e-12)
    t_h = (jnp.dot(h, whx_ref[...]) + jnp.dot(m_h, whm_ref[...])
           + norm * v[1:2, :16] + v[2:3, :16])
    h_upd = jnp.dot(_silu(t_h), wh2_ref[...]) + v[3:4, :]
    if has_sc:
        sc = jnp.dot(h, wsc_ref[...]) + v[4:5, :]
    else:
        sc = h
    y = jnp.maximum(sc + h_upd, 0.0)
    mean = jnp.mean(y, axis=1, keepdims=True)
    var = jnp.mean((y - mean) ** 2, axis=1, keepdims=True)
    o_ref[...] = v[5:6, :] * (y - mean) / jnp.sqrt(var + 1e-5) + v[6:7, :]


def _node_phase(h, acc0, acc1, we2, whx, whm, wh2, wsc, vpack, has_sc):
    n, cin = h.shape
    grid = (n // NBLK,)
    return pl.pallas_call(
        functools.partial(_node_body, has_sc),
        grid=grid,
        in_specs=[
            pl.BlockSpec((NBLK, cin), lambda i: (i, 0)),
            pl.BlockSpec((NBLK, MW), lambda i: (i, 0)),
            pl.BlockSpec((NBLK, MW), lambda i: (i, 0)),
            pl.BlockSpec((32, 16), lambda i: (0, 0)),
            pl.BlockSpec((cin, 16), lambda i: (0, 0)),
            pl.BlockSpec((16, 16), lambda i: (0, 0)),
            pl.BlockSpec((16, 64), lambda i: (0, 0)),
            pl.BlockSpec((cin, 64), lambda i: (0, 0)),
            pl.BlockSpec((8, 64), lambda i: (0, 0)),
        ],
        out_specs=pl.BlockSpec((NBLK, 64), lambda i: (i, 0)),
        out_shape=jax.ShapeDtypeStruct((n, 64), jnp.float32),
    )(h, acc0, acc1, we2, whx, whm, wh2, wsc, vpack)


def _pool_body(nsteps, h_ref, pv_ref, b_ref, wp_ref, wo1_ref, wo2_ref,
               v_ref, s_ref, lat_ref, mu_ref, ent_ref, p_acc, e_acc):
    i = pl.program_id(0)
    h = h_ref[...]
    v = v_ref[...]
    rb = h.shape[0]

    @pl.when(i == 0)
    def _():
        p_acc[...] = jnp.zeros_like(p_acc)
        e_acc[...] = jnp.zeros_like(e_acc)

    logits = jnp.dot(h, wp_ref[...]) + v[0:1, :16]
    z = logits * (1.0 / TAU)
    z = z - jnp.max(z, axis=1, keepdims=True)
    p = jnp.exp(z)
    s = p / jnp.sum(p, axis=1, keepdims=True)
    s_ref[...] = s

    ent = jnp.sum(s * jnp.log(s + 1e-8))
    e_acc[...] = e_acc[...] + ent

    feat = jnp.concatenate(
        [h, pv_ref[...][:, :2], jnp.ones((rb, 1), jnp.float32),
         jnp.zeros((rb, 5), jnp.float32)], axis=1)
    bcol = b_ref[...]
    for b in range(NB):
        mask = (bcol == float(b)).astype(jnp.float32)
        ms = s * mask
        contrib = lax.dot_general(ms, feat, (((0,), (0,)), ((), ())))
        p_acc[b * 16:(b + 1) * 16, :] = p_acc[b * 16:(b + 1) * 16, :] + contrib

    @pl.when(i == nsteps - 1)
    def _():
        tot = p_acc[...]
        sw = tot[:, 66:67] + 1e-8
        pooled = tot[:, :64] / sw
        hid = jnp.maximum(jnp.dot(pooled, wo1_ref[...]) + v[1:2, :32], 0.0)
        lat_ref[...] = jnp.dot(hid, wo2_ref[...]) + v[2:3, :32]
        mu = tot[:, 64:66] / sw
        mu_ref[...] = jnp.concatenate(
            [mu, jnp.zeros((mu.shape[0], 6), jnp.float32)], axis=1)
        ent_ref[...] = e_acc[...] * (-1.0 / (nsteps * rb))


def _pooling(h, pv, batchf, wp, wo1, wo2, vpack):
    n = h.shape[0]
    nsteps = n // NBLK
    return pl.pallas_call(
        functools.partial(_pool_body, nsteps),
        grid=(nsteps,),
        in_specs=[
            pl.BlockSpec((NBLK, 64), lambda i: (i, 0)),
            pl.BlockSpec((NBLK, 4), lambda i: (i, 0)),
            pl.BlockSpec((NBLK, 1), lambda i: (i, 0)),
            pl.BlockSpec((64, 16), lambda i: (0, 0)),
            pl.BlockSpec((64, 32), lambda i: (0, 0)),
            pl.BlockSpec((32, 32), lambda i: (0, 0)),
            pl.BlockSpec((4, 32), lambda i: (0, 0)),
        ],
        out_specs=[
            pl.BlockSpec((NBLK, 16), lambda i: (i, 0)),
            pl.BlockSpec((NB * 16, 32), lambda i: (0, 0)),
            pl.BlockSpec((NB * 16, 8), lambda i: (0, 0)),
            pl.BlockSpec((8, 128), lambda i: (0, 0)),
        ],
        out_shape=[
            jax.ShapeDtypeStruct((n, 16), jnp.float32),
            jax.ShapeDtypeStruct((NB * 16, 32), jnp.float32),
            jax.ShapeDtypeStruct((NB * 16, 8), jnp.float32),
            jax.ShapeDtypeStruct((8, 128), jnp.float32),
        ],
        scratch_shapes=[
            pltpu.VMEM((NB * 16, 72), jnp.float32),
            pltpu.VMEM((8, 128), jnp.float32),
        ],
    )(h, pv, batchf, wp, wo1, wo2, vpack)


def _layer_weights(p, cin):
    we1, be1 = p["phi_e1"]["w"], p["phi_e1"]["b"]
    wv1, bv1 = p["phi_v1"]["w"], p["phi_v1"]["b"]
    w_src = jnp.concatenate([we1[cin:2 * cin], wv1[cin:2 * cin]], axis=1)
    w_dst = jnp.concatenate([we1[:cin], wv1[:cin]], axis=1)
    z16 = jnp.zeros((16,), jnp.float32)

    def r32(v):
        return jnp.pad(v, (0, 32 - v.shape[0]))

    wpack = jnp.stack([
        we1[2 * cin], we1[2 * cin + 1], be1,
        r32(wv1[2 * cin]), r32(wv1[2 * cin + 1]), r32(bv1),
        r32(p["phi_v2"]["w"][:, 0]), r32(jnp.pad(p["phi_v2"]["b"], (0, 15))),
    ])
    wh1 = p["phi_h1"]["w"]
    whx, whm, whn = wh1[:cin], wh1[cin:cin + 16], wh1[cin + 16]
    del z16
    return w_src, w_dst, wpack, whx, whm, whn


def _vpack_node(p, ln, whn, cin):
    def r64(v):
        return jnp.pad(v, (0, 64 - v.shape[0]))

    if "shortcut" in p:
        bsc = p["shortcut"]["b"]
        wsc = p["shortcut"]["w"]
    else:
        bsc = jnp.zeros((64,), jnp.float32)
        wsc = jnp.zeros((cin, 64), jnp.float32)
    rows = [
        r64(p["phi_e2"]["b"]), r64(whn), r64(p["phi_h1"]["b"]),
        p["phi_h2"]["b"], bsc, ln["g"], ln["b"],
        jnp.zeros((64,), jnp.float32),
    ]
    return jnp.stack(rows), wsc


def _egnn_layer_pipe(p, ln, h, pv, src, dst, zeros_n):
    cin = h.shape[1]
    w_src, w_dst, wpack, whx, whm, whn = _layer_weights(p, cin)
    vpack, wsc = _vpack_node(p, ln, whn, cin)
    t_src, t_dst = _pre_tables(h, pv, w_src, w_dst)
    esum = _sc_gather(t_src, t_dst, src, dst)
    m_rows = _edge_dense(esum, wpack)
    parts = _sc_scatter(m_rows, dst, zeros_n)
    n = h.shape[0]
    return _node_phase(h, parts[0, :n], parts[1, :n], p["phi_e2"]["w"], whx, whm,
                       p["phi_h2"]["w"], wsc, vpack, "shortcut" in p)


def kernel(x, edge_index, batch, params):
    n = x.shape[0]
    src = edge_index[0]
    dst = edge_index[1]
    pv = x[:, :4]
    npad = ((n + 127) // 128) * 128
    zeros_n = jnp.zeros((npad, MW), jnp.float32)
    batchf = batch.astype(jnp.float32).reshape(n, 1)

    h = _egnn_layer_pipe(params["gnn1"], params["ln1"], x, pv, src, dst,
                         zeros_n)
    h = _egnn_layer_pipe(params["gnn2"], params["ln2"], h, pv, src, dst,
                         zeros_n)

    def r32(v):
        return jnp.pad(v, (0, 32 - v.shape[0]))

    vpack = jnp.stack([
        r32(params["pool"]["b"]), params["out1"]["b"], params["out2"]["b"],
        jnp.zeros((32,), jnp.float32),
    ])
    s, lat, mu8, entv = _pooling(h, pv, batchf, params["pool"]["w"],
                                 params["out1"]["w"], params["out2"]["w"],
                                 vpack)
    latent = lat.reshape(NB, 16, 32)
    mu = mu8[:, :2].reshape(NB, 16, 2)
    assign_loss = entv[0, 0]
    return latent, s, assign_loss, mu
